# baseline (speedup 1.0000x reference)
import jax
import jax.numpy as jnp
from jax.experimental import pallas as pl

_NBITS = 96
_HIDDEN = 32
_CH = 128


def _body(msg_ref, even_ref, odd_ref, lat_ref, out_ref):
    even = even_ref[...]
    odd = odd_ref[...]
    diff = odd - even
    base = jnp.sum(even, axis=0)
    msg_f = msg_ref[...].astype(jnp.float32)
    aux = jax.lax.dot_general(
        msg_f, diff, (((1,), (0,)), ((), ())),
        preferred_element_type=jnp.float32) + base[None, :]
    out_ref[:, :_CH, :] = lat_ref[...]
    bblk = msg_f.shape[0]
    hw = lat_ref.shape[2]
    out_ref[:, _CH:, :] = jnp.broadcast_to(aux[:, :, None], (bblk, _HIDDEN, hw))


def kernel(latents, msg, msg_embeddings):
    batch, ch, s1, s2 = latents.shape
    hw = s1 * s2
    lat = latents.reshape(batch, ch, hw)
    even = msg_embeddings[0::2]
    odd = msg_embeddings[1::2]

    bblk = 8
    grid = (batch // bblk,)
    out = pl.pallas_call(
        _body,
        grid=grid,
        in_specs=[
            pl.BlockSpec((bblk, _NBITS), lambda b: (b, 0)),
            pl.BlockSpec((_NBITS, _HIDDEN), lambda b: (0, 0)),
            pl.BlockSpec((_NBITS, _HIDDEN), lambda b: (0, 0)),
            pl.BlockSpec((bblk, ch, hw), lambda b: (b, 0, 0)),
        ],
        out_specs=pl.BlockSpec((bblk, ch + _HIDDEN, hw), lambda b: (b, 0, 0)),
        out_shape=jax.ShapeDtypeStruct((batch, ch + _HIDDEN, hw), jnp.float32),
    )(msg, even, odd, lat)
    return out.reshape(batch, ch + _HIDDEN, s1, s2)
